# Initial kernel scaffold; baseline (speedup 1.0000x reference)
import jax
import jax.numpy as jnp
from jax.experimental import pallas as pl


def kernel(pose_indices, image, emb_table, W3, b3):
    raise NotImplementedError("write your pallas kernel here")


# speedup vs baseline: 13.6868x; 13.6868x over previous
import functools

import jax
import jax.numpy as jnp
from jax import lax
from jax.experimental import pallas as pl
from jax.experimental.pallas import tpu as pltpu
from jax.experimental.pallas import tpu_sc as plsc

_B = 16384
_J = 12
_V = 160
_D = 8
_NC = 2
_NS = 16
_NW = _NC * _NS
_BPW = _B // _NW
_RB = _BPW // 16


def _table_body(emb_ref, w3_ref, b3_ref, l_ref):
    e = jnp.maximum(emb_ref[...], 0.0)
    w = w3_ref[...]
    for j in range(_J):
        blk = w[:, _D * j:_D * (j + 1)]
        lj = lax.dot_general(blk, e, (((1,), (1,)), ((), ())),
                             preferred_element_type=jnp.float32)
        if j == 0:
            lj = lj + b3_ref[...]
        l_ref[:, _V * j:_V * (j + 1)] = lj


_build_table = pl.pallas_call(
    _table_body,
    out_shape=jax.ShapeDtypeStruct((3, _J * _V), jnp.float32),
)


@functools.partial(
    pl.kernel,
    out_type=jax.ShapeDtypeStruct((_B * 3,), jnp.float32),
    mesh=plsc.VectorSubcoreMesh(core_axis_name="c", subcore_axis_name="s"),
    compiler_params=pltpu.CompilerParams(needs_layout_passes=False),
    scratch_types=[
        pltpu.VMEM((_J * _V,), jnp.float32),
        pltpu.VMEM((_J * _V,), jnp.float32),
        pltpu.VMEM((_J * _V,), jnp.float32),
        pltpu.VMEM((_BPW * _J,), jnp.int32),
        pltpu.VMEM((_BPW * 3,), jnp.float32),
    ],
)
def _sc_lookup(l_hbm, idx_hbm, out_hbm, l0_v, l1_v, l2_v, idx_v, out_v):
    wid = lax.axis_index("s") * _NC + lax.axis_index("c")
    base = wid * _BPW
    pltpu.sync_copy(idx_hbm.at[pl.ds(base * _J, _BPW * _J)], idx_v)
    pltpu.sync_copy(l_hbm.at[pl.ds(0, _J * _V)], l0_v)
    pltpu.sync_copy(l_hbm.at[pl.ds(_J * _V, _J * _V)], l1_v)
    pltpu.sync_copy(l_hbm.at[pl.ds(2 * _J * _V, _J * _V)], l2_v)

    lanes = jax.lax.iota(jnp.int32, 16)
    lanes_j = lanes * _J
    lanes_3 = lanes * 3

    def body(rb, carry):
        pbase = rb * (16 * _J)
        acc0 = jnp.zeros((16,), jnp.float32)
        acc1 = jnp.zeros((16,), jnp.float32)
        acc2 = jnp.zeros((16,), jnp.float32)
        for j in range(_J):
            pidx = lanes_j + (pbase + j)
            vj = plsc.load_gather(idx_v, [pidx])
            fidx = vj + (j * _V)
            acc0 = acc0 + plsc.load_gather(l0_v, [fidx])
            acc1 = acc1 + plsc.load_gather(l1_v, [fidx])
            acc2 = acc2 + plsc.load_gather(l2_v, [fidx])
        obase = rb * 48
        plsc.store_scatter(out_v, [lanes_3 + obase], acc0)
        plsc.store_scatter(out_v, [lanes_3 + (obase + 1)], acc1)
        plsc.store_scatter(out_v, [lanes_3 + (obase + 2)], acc2)
        return carry

    lax.fori_loop(0, _RB, body, 0)
    pltpu.sync_copy(out_v, out_hbm.at[pl.ds(base * 3, _BPW * 3)])


def kernel(pose_indices, image, emb_table, W3, b3):
    del image
    l_table = _build_table(emb_table, W3, b3.reshape(3, 1))
    out_flat = _sc_lookup(l_table.reshape(-1),
                          pose_indices.astype(jnp.int32).reshape(-1))
    return out_flat.reshape(_B, 3)
